# baseline (speedup 1.0000x reference)
import functools

import jax
import jax.numpy as jnp
from jax import lax
from jax.experimental import pallas as pl
from jax.experimental.pallas import tpu as pltpu
from jax.experimental.pallas import tpu_sc as plsc

N_NODE = 50000
IN_DIM = 128
OUT_DIM = 128
H = 8
DK = 16
E = 300000
SQRT_DK = 4.0

NC = 2
NS = 16
L = 16
E_PAD = 300032
ES = E_PAD // NS
SB = 4688
NBLK = ES // SB
NPASS = 4
N_OUT_PAD = 50048
CHUNK = 6256
ACC_ROWS = 6400
DUMMY_LOCAL = 6256
DEN_ROWS = ACC_ROWS // 8
DEN_ZCOPIES = DEN_ROWS // L
B = 64
SEL_CAP = SB + B
ROWS_PER_TILE = ACC_ROWS // NS
DRAIN_Q = 392
NB = 56
DRAIN_LAST_BLOCKS = 6
DRAIN_LAST_TAIL = 40

BR = 1000
GRID = N_NODE // BR


def _proj_body(ha_ref, hp_ref, tp_ref, wkw_ref, wvw_ref, wkc_ref, wvc_ref,
               wq_ref, bkw_ref, bvw_ref, bkc_ref, bvc_ref, bq_ref,
               kvw_ref, kvc_ref, q_ref):
    ha = ha_ref[...]
    hp = hp_ref[...]
    tp = tp_ref[...]
    f32 = jnp.float32
    bf16 = jnp.bfloat16
    kvw_ref[:, :OUT_DIM] = (jnp.dot(ha, wkw_ref[...], preferred_element_type=f32)
                            + bkw_ref[...]).astype(bf16)
    kvw_ref[:, OUT_DIM:] = (jnp.dot(ha, wvw_ref[...], preferred_element_type=f32)
                            + bvw_ref[...]).astype(bf16)
    kvc_ref[:, :OUT_DIM] = (jnp.dot(hp, wkc_ref[...], preferred_element_type=f32)
                            + bkc_ref[...]).astype(bf16)
    kvc_ref[:, OUT_DIM:] = (jnp.dot(hp, wvc_ref[...], preferred_element_type=f32)
                            + bvc_ref[...]).astype(bf16)
    q_ref[...] = jnp.dot(tp, wq_ref[...], preferred_element_type=f32) + bq_ref[...]


def _run_projections(h_author, h_paper, t_paper, wkw, wvw, wkc, wvc, wq,
                     bkw, bvw, bkc, bvc, bq):
    row_spec = pl.BlockSpec((BR, IN_DIM), lambda i: (i, 0))
    w_spec = pl.BlockSpec((IN_DIM, OUT_DIM), lambda i: (0, 0))
    b_spec = pl.BlockSpec((1, OUT_DIM), lambda i: (0, 0))
    kv_spec = pl.BlockSpec((BR, 2 * OUT_DIM), lambda i: (i, 0))
    kv_sd = jax.ShapeDtypeStruct((N_NODE, 2 * OUT_DIM), jnp.bfloat16)
    q_sd = jax.ShapeDtypeStruct((N_NODE, OUT_DIM), jnp.float32)
    return pl.pallas_call(
        _proj_body,
        grid=(GRID,),
        in_specs=[row_spec, row_spec, row_spec,
                  w_spec, w_spec, w_spec, w_spec, w_spec,
                  b_spec, b_spec, b_spec, b_spec, b_spec],
        out_specs=[kv_spec, kv_spec, row_spec],
        out_shape=[kv_sd, kv_sd, q_sd],
    )(h_author, h_paper, t_paper, wkw, wvw, wkc, wvc, wq,
      bkw, bvw, bkc, bvc, bq)


def _edge_kernel_body(src_hbm, dst_hbm, kvtab, qtab, mu_out,
                      mu_acc, den_acc, srcbuf, dstbuf, sel_pk,
                      kvb0, qwb0, vst0, kvb1, qwb1, vst1,
                      idxsrc0, idxdst0, idxstage0, idxden0,
                      idxsrc1, idxdst1, idxstage1, idxden1, semg0, semg1):
    cid = lax.axis_index("c")
    sid = lax.axis_index("s")
    iota = lax.iota(jnp.int32, L)
    zrow = jnp.zeros((L,), jnp.float32)
    mhi = jnp.int32(-65536)

    def bc_lo(w):
        return lax.bitcast_convert_type(lax.shift_left(w, 16), jnp.float32)

    def bc_hi(w):
        return lax.bitcast_convert_type(w & mhi, jnp.float32)

    def pass_body(p, _):
        lo = (NC * p + cid) * CHUNK

        def zrow_step(r, _):
            for cc in range(OUT_DIM // L):
                vst0[r, cc * L:(cc + 1) * L] = zrow
            return 0

        lax.fori_loop(0, L, zrow_step, 0)
        zsrc = vst0.at[pl.ds(0, L)]

        def zcopy_step(z, _):
            base = sid * ROWS_PER_TILE + z * L
            pltpu.sync_copy(zsrc, mu_acc.at[pl.ds(base, L)])
            return 0

        lax.fori_loop(0, ROWS_PER_TILE // L, zcopy_step, 0)
        for z in range(DEN_ZCOPIES // NS + 1):
            j = sid + z * NS

            @pl.when(j < DEN_ZCOPIES)
            def _():
                pltpu.sync_copy(zsrc, den_acc.at[pl.ds(j * L, L)])
        plsc.subcore_barrier()

        def block_body(b, _):
            ebase = sid * ES + b * SB
            pltpu.sync_copy(dst_hbm.at[pl.ds(ebase, SB)], dstbuf)
            pltpu.sync_copy(src_hbm.at[pl.ds(ebase, SB)], srcbuf)

            def sel_step(i, off):
                d = dstbuf[pl.ds(i * L, L)]
                s = srcbuf[pl.ds(i * L, L)]
                m = (d >= lo) & (d < lo + CHUNK)
                mi = m.astype(jnp.int32)
                cs = plsc.cumsum(mi)
                pos = off + cs - 1
                packed = lax.shift_left(d, 16) | s
                plsc.store_scatter(sel_pk, [pos], packed, mask=m)
                return off + cs[L - 1]

            nsel = lax.fori_loop(0, SB // L, sel_step, jnp.int32(0))

            dummy_pk = lax.shift_left(jnp.full((L,), 0, jnp.int32) + lo, 16)
            for j in range(B // L):
                ppos = nsel + j * L + iota
                plsc.store_scatter(sel_pk, [ppos], dummy_pk)
            nchunks = nsel // B + 1

            def gathers(idxsrcS, idxdstS, kvbS, qwbS, semS):
                return (pltpu.make_async_copy(kvtab.at[idxsrcS.at[0]], kvbS, semS),
                        pltpu.make_async_copy(qtab.at[idxdstS.at[0]], qwbS, semS))

            def prefetch(c, slot):
                (idxsrcS, idxdstS, idxstageS, idxdenS,
                 kvbS, qwbS, vstS, semS) = slot
                base = c * B
                for j in range(B // L):
                    gpos = base + j * L + iota
                    w = sel_pk[pl.ds(base + j * L, L)]
                    sv_ = w & 0xFFFF
                    dv = lax.shift_right_logical(w, 16)
                    idxsrcS[0, j * L:(j + 1) * L] = sv_
                    idxdstS[0, j * L:(j + 1) * L] = dv
                    local = jnp.where(gpos < nsel, dv - lo, DUMMY_LOCAL)
                    idxstageS[0, j * L:(j + 1) * L] = local
                    idxdenS[0, j * L:(j + 1) * L] = jnp.right_shift(local, 3)
                for cp in gathers(idxsrcS, idxdstS, kvbS, qwbS, semS):
                    cp.start()

            def compute(slot):
                (idxsrcS, idxdstS, idxstageS, idxdenS,
                 kvbS, qwbS, vstS, semS) = slot
                for cp in gathers(idxsrcS, idxdstS, kvbS, qwbS, semS):
                    cp.wait()

                def edge_group(g, _):
                    locvec = idxstageS[0, pl.ds(g * L, L)]
                    for e16 in range(L):
                        e = g * L + e16
                        sv = jnp.zeros((L,), jnp.float32)
                        for hp in range(H // 2):
                            kw = kvbS[e, hp * L:(hp + 1) * L]
                            qlo = qwbS[e, (2 * hp) * L:(2 * hp + 1) * L]
                            qhi = qwbS[e, (2 * hp + 1) * L:(2 * hp + 2) * L]
                            prod = bc_lo(kw) * qlo + bc_hi(kw) * qhi
                            cs = plsc.cumsum(prod)
                            sa = cs[7]
                            sb = cs[15] - cs[7]
                            sv = jnp.where(iota == 2 * hp, sa, sv)
                            sv = jnp.where(iota == 2 * hp + 1, sb, sv)
                        ex = jnp.exp(sv)
                        for blk in range(8):
                            qwbS[e, blk * L:(blk + 1) * L] = zrow
                        boff = (locvec[e16] & 7) * L
                        qwbS[e, pl.ds(boff, L)] = ex
                        for hp in range(H // 2):
                            vw = kvbS[e, OUT_DIM // 2 + hp * L:
                                      OUT_DIM // 2 + (hp + 1) * L]
                            sc = jnp.where(iota < 8, ex[2 * hp], ex[2 * hp + 1])
                            vstS[e, (2 * hp) * L:(2 * hp + 1) * L] = bc_lo(vw) * sc
                            vstS[e, (2 * hp + 1) * L:(2 * hp + 2) * L] = bc_hi(vw) * sc
                    return 0

                lax.fori_loop(0, B // L, edge_group, 0)

                pltpu.sync_copy(vstS, mu_acc.at[idxstageS.at[0]], add=True)
                pltpu.sync_copy(qwbS, den_acc.at[idxdenS.at[0]], add=True)

            slot0 = (idxsrc0, idxdst0, idxstage0, idxden0,
                     kvb0, qwb0, vst0, semg0)
            slot1 = (idxsrc1, idxdst1, idxstage1, idxden1,
                     kvb1, qwb1, vst1, semg1)
            prefetch(jnp.int32(0), slot0)

            def pair_body(t, _):
                c0 = 2 * t

                @pl.when(c0 + 1 < nchunks)
                def _():
                    prefetch(c0 + 1, slot1)

                compute(slot0)

                @pl.when(c0 + 2 < nchunks)
                def _():
                    prefetch(c0 + 2, slot0)

                @pl.when(c0 + 1 < nchunks)
                def _():
                    compute(slot1)
                return 0

            lax.fori_loop(0, (nchunks + 1) // 2, pair_body, 0)
            return 0

        lax.fori_loop(0, NBLK, block_body, 0)
        plsc.subcore_barrier()

        def norm_drain(start, dstart, nrows, ndrows):
            cp = pltpu.async_copy(mu_acc.at[pl.ds(start, nrows)],
                                  vst0.at[pl.ds(0, nrows)], semg0)
            pltpu.sync_copy(den_acc.at[pl.ds(dstart, ndrows)],
                            qwb0.at[pl.ds(0, ndrows)])
            cp.wait()

            def node_step(n, _):
                r = jnp.right_shift(n, 3)
                c0 = (n & 7) * L
                dvec = jnp.maximum(qwb0[r, pl.ds(c0, L)], 1e-37)
                for hp in range(H // 2):
                    dsel = jnp.where(iota < 8, dvec[2 * hp], dvec[2 * hp + 1])
                    for bb in (2 * hp, 2 * hp + 1):
                        mh = vst0[n, bb * L:(bb + 1) * L]
                        vst0[n, bb * L:(bb + 1) * L] = mh / dsel
                return 0

            lax.fori_loop(0, nrows, node_step, 0)
            pltpu.sync_copy(vst0.at[pl.ds(0, nrows)],
                            mu_out.at[pl.ds(lo + start, nrows)])

        @pl.when(sid < NS - 1)
        def _():
            for c in range(DRAIN_Q // NB):
                norm_drain(sid * DRAIN_Q + c * NB,
                           sid * (DRAIN_Q // 8) + c * (NB // 8), NB, NB // 8)

        @pl.when(sid == NS - 1)
        def _():
            for c in range(DRAIN_LAST_BLOCKS):
                norm_drain((NS - 1) * DRAIN_Q + c * NB,
                           (NS - 1) * (DRAIN_Q // 8) + c * (NB // 8), NB, NB // 8)
            norm_drain((NS - 1) * DRAIN_Q + DRAIN_LAST_BLOCKS * NB,
                       (NS - 1) * (DRAIN_Q // 8) + DRAIN_LAST_BLOCKS * (NB // 8),
                       DRAIN_LAST_TAIL, DRAIN_LAST_TAIL // 8)
        plsc.subcore_barrier()
        return 0

    lax.fori_loop(0, NPASS, pass_body, 0)


def _run_edges(src, dst, kvtab, qtab):
    mesh = plsc.VectorSubcoreMesh(core_axis_name="c", subcore_axis_name="s",
                                  num_cores=NC, num_subcores=NS)
    run = pl.kernel(
        _edge_kernel_body,
        out_type=jax.ShapeDtypeStruct((N_OUT_PAD, OUT_DIM), jnp.float32),
        mesh=mesh,
        scratch_types=[
            pltpu.VMEM_SHARED((ACC_ROWS, OUT_DIM), jnp.float32),
            pltpu.VMEM_SHARED((DEN_ROWS, OUT_DIM), jnp.float32),
            pltpu.VMEM((SB,), jnp.int32),
            pltpu.VMEM((SB,), jnp.int32),
            pltpu.VMEM((SEL_CAP,), jnp.int32),
            pltpu.VMEM((B, OUT_DIM), jnp.int32),
            pltpu.VMEM((B, OUT_DIM), jnp.float32),
            pltpu.VMEM((B, OUT_DIM), jnp.float32),
            pltpu.VMEM((B, OUT_DIM), jnp.int32),
            pltpu.VMEM((B, OUT_DIM), jnp.float32),
            pltpu.VMEM((B, OUT_DIM), jnp.float32),
            pltpu.VMEM((1, B), jnp.int32),
            pltpu.VMEM((1, B), jnp.int32),
            pltpu.VMEM((1, B), jnp.int32),
            pltpu.VMEM((1, B), jnp.int32),
            pltpu.VMEM((1, B), jnp.int32),
            pltpu.VMEM((1, B), jnp.int32),
            pltpu.VMEM((1, B), jnp.int32),
            pltpu.VMEM((1, B), jnp.int32),
            pltpu.SemaphoreType.DMA,
            pltpu.SemaphoreType.DMA,
        ],
        compiler_params=pltpu.CompilerParams(needs_layout_passes=False),
    )
    return run(src, dst, kvtab, qtab)


def _final_body(muw_ref, muc_ref, tp_ref, wa_ref, ba_ref,
                g_ref, b_ref, al_ref, out_ref):
    mu = (muw_ref[...] + muc_ref[...]) * 0.5
    trans = jnp.dot(mu, wa_ref[...], preferred_element_type=jnp.float32) + ba_ref[...]
    alpha = al_ref[0, 0]
    out = trans * alpha + (1.0 - alpha) * tp_ref[...]
    mean = jnp.mean(out, axis=1, keepdims=True)
    d = out - mean
    var = jnp.mean(d * d, axis=1, keepdims=True)
    out_ref[...] = d * lax.rsqrt(var + 1e-5) * g_ref[...] + b_ref[...]


def _run_final(mu_w, mu_c, t_paper, waT, ba1, g1, b1, alpha):
    row_spec = pl.BlockSpec((BR, OUT_DIM), lambda i: (i, 0))
    w_spec = pl.BlockSpec((OUT_DIM, OUT_DIM), lambda i: (0, 0))
    b_spec = pl.BlockSpec((1, OUT_DIM), lambda i: (0, 0))
    s_spec = pl.BlockSpec((1, 1), lambda i: (0, 0))
    return pl.pallas_call(
        _final_body,
        grid=(GRID,),
        in_specs=[row_spec, row_spec, row_spec,
                  w_spec, b_spec, b_spec, b_spec, s_spec],
        out_specs=row_spec,
        out_shape=jax.ShapeDtypeStruct((N_NODE, OUT_DIM), jnp.float32),
    )(mu_w, mu_c, t_paper, waT, ba1, g1, b1, alpha)


def kernel(h_author, h_paper, t_author, t_paper, edge_index_writes,
           edge_index_cites, Wk, bk, Wq, bq, Wv, bv, Wa, ba, ln_scale,
           ln_bias, relation_pri, relation_att, relation_msg, skip):
    f32 = jnp.float32

    def fold(WT, bvec, rel, scale):
        wt = jnp.einsum('dhi,hij->dhj', WT.reshape(IN_DIM, H, DK), rel)
        bf = jnp.einsum('hi,hij->hj', bvec.reshape(H, DK), rel)
        wt = wt * scale[None, :, None]
        bf = bf * scale[:, None]
        return wt.reshape(IN_DIM, OUT_DIM), bf.reshape(1, OUT_DIM)

    pri_w = relation_pri[0] / SQRT_DK
    pri_c = relation_pri[1] / SQRT_DK
    ones_h = jnp.ones((H,), f32)
    wkw, bkw = fold(Wk[0].T, bk[0], relation_att[0], pri_w)
    wkc, bkc = fold(Wk[1].T, bk[1], relation_att[1], pri_c)
    wvw, bvw = fold(Wv[0].T, bv[0], relation_msg[0], ones_h)
    wvc, bvc = fold(Wv[1].T, bv[1], relation_msg[1], ones_h)
    perm = jnp.array([16 * b + 2 * j if b % 2 == 0 else 16 * b - 16 + 2 * j + 1
                      for b in range(H) for j in range(L)], jnp.int32)
    wq = Wq[1].T[:, perm]
    bq1 = bq[1].reshape(1, OUT_DIM)[:, perm]

    kv_w, kv_c, q = _run_projections(
        h_author, h_paper, t_paper, wkw, wvw, wkc, wvc, wq,
        bkw, bvw, bkc, bvc, bq1)

    def pack_i32(x):
        n, c2 = x.shape
        return lax.bitcast_convert_type(x.reshape(n, c2 // 2, 2), jnp.int32)

    kvt_w = pack_i32(kv_w)
    kvt_c = pack_i32(kv_c)
    qt = q

    def pad_edges(ei):
        src = jnp.concatenate([ei[0], jnp.zeros((E_PAD - E,), jnp.int32)])
        dst = jnp.concatenate([ei[1], jnp.full((E_PAD - E,), -1, jnp.int32)])
        return src, dst

    src_w, dst_w = pad_edges(edge_index_writes)
    src_c, dst_c = pad_edges(edge_index_cites)

    mu_w = _run_edges(src_w, dst_w, kvt_w, qt)
    mu_c = _run_edges(src_c, dst_c, kvt_c, qt)

    waT = Wa[1].T[perm, :]
    alpha = jax.nn.sigmoid(skip[1]).reshape(1, 1).astype(f32)
    return _run_final(mu_w, mu_c, t_paper, waT,
                      ba[1].reshape(1, OUT_DIM), ln_scale[1].reshape(1, OUT_DIM),
                      ln_bias[1].reshape(1, OUT_DIM), alpha)


# speedup vs baseline: 1.3866x; 1.3866x over previous
import functools

import jax
import jax.numpy as jnp
from jax import lax
from jax.experimental import pallas as pl
from jax.experimental.pallas import tpu as pltpu
from jax.experimental.pallas import tpu_sc as plsc

N_NODE = 50000
IN_DIM = 128
OUT_DIM = 128
H = 8
DK = 16
E = 300000
SQRT_DK = 4.0

NC = 2
NS = 16
L = 16
E_PAD = 300032
ES = E_PAD // NS
SB = 4688
NBLK = ES // SB
NPASS = 4
N_OUT_PAD = 50048
CHUNK = 6256
ACC_ROWS = 6400
DUMMY_LOCAL = 6256
DEN_ROWS = ACC_ROWS // 8
DEN_ZCOPIES = DEN_ROWS // L
B = 64
SEL_CAP = SB + B
ROWS_PER_TILE = ACC_ROWS // NS
DRAIN_Q = 392
NB = 56
DRAIN_LAST_TAIL = 40

BR = 1000
GRID = N_NODE // BR


def _proj_body(ha_ref, hp_ref, tp_ref, wkw_ref, wvw_ref, wkc_ref, wvc_ref,
               wq_ref, bkw_ref, bvw_ref, bkc_ref, bvc_ref, bq_ref,
               kw_ref, vw_ref, kc_ref, vc_ref, q_ref):
    ha = ha_ref[...]
    hp = hp_ref[...]
    tp = tp_ref[...]
    f32 = jnp.float32
    kw_ref[...] = jnp.dot(ha, wkw_ref[...], preferred_element_type=f32) + bkw_ref[...]
    vw_ref[...] = jnp.dot(ha, wvw_ref[...], preferred_element_type=f32) + bvw_ref[...]
    kc_ref[...] = jnp.dot(hp, wkc_ref[...], preferred_element_type=f32) + bkc_ref[...]
    vc_ref[...] = jnp.dot(hp, wvc_ref[...], preferred_element_type=f32) + bvc_ref[...]
    q_ref[...] = jnp.dot(tp, wq_ref[...], preferred_element_type=f32) + bq_ref[...]


def _run_projections(h_author, h_paper, t_paper, wkw, wvw, wkc, wvc, wq,
                     bkw, bvw, bkc, bvc, bq):
    row_spec = pl.BlockSpec((BR, IN_DIM), lambda i: (i, 0))
    w_spec = pl.BlockSpec((IN_DIM, OUT_DIM), lambda i: (0, 0))
    b_spec = pl.BlockSpec((1, OUT_DIM), lambda i: (0, 0))
    out_sd = jax.ShapeDtypeStruct((N_NODE, OUT_DIM), jnp.float32)
    return pl.pallas_call(
        _proj_body,
        grid=(GRID,),
        in_specs=[row_spec, row_spec, row_spec,
                  w_spec, w_spec, w_spec, w_spec, w_spec,
                  b_spec, b_spec, b_spec, b_spec, b_spec],
        out_specs=[row_spec] * 5,
        out_shape=[out_sd] * 5,
    )(h_author, h_paper, t_paper, wkw, wvw, wkc, wvc, wq,
      bkw, bvw, bkc, bvc, bq)


def _edge_kernel_body(src_hbm, dst_hbm, ktab, qtab, vtab, mu_out,
                      mu_acc, den_acc, srcbuf, dstbuf, sel_pk,
                      kbuf0, qbuf0, vbuf0, kbuf1, qbuf1, vbuf1,
                      idxsrc0, idxdst0, idxstage0, idxden0,
                      idxsrc1, idxdst1, idxstage1, idxden1, semg0, semg1):
    cid = lax.axis_index("c")
    sid = lax.axis_index("s")
    iota = lax.iota(jnp.int32, L)
    zrow = jnp.zeros((L,), jnp.float32)

    def pass_body(p, _):
        lo = (NC * p + cid) * CHUNK

        def zrow_step(r, _):
            for cc in range(OUT_DIM // L):
                kbuf0[r, cc * L:(cc + 1) * L] = zrow
            return 0

        lax.fori_loop(0, L, zrow_step, 0)
        zsrc = kbuf0.at[pl.ds(0, L)]

        def zcopy_step(z, _):
            base = sid * ROWS_PER_TILE + z * L
            pltpu.sync_copy(zsrc, mu_acc.at[pl.ds(base, L)])
            return 0

        lax.fori_loop(0, ROWS_PER_TILE // L, zcopy_step, 0)
        for z in range(DEN_ZCOPIES // NS + 1):
            j = sid + z * NS

            @pl.when(j < DEN_ZCOPIES)
            def _():
                pltpu.sync_copy(zsrc, den_acc.at[pl.ds(j * L, L)])
        plsc.subcore_barrier()

        def block_body(b, _):
            ebase = sid * ES + b * SB
            pltpu.sync_copy(dst_hbm.at[pl.ds(ebase, SB)], dstbuf)
            pltpu.sync_copy(src_hbm.at[pl.ds(ebase, SB)], srcbuf)

            def sel_step(i, off):
                d = dstbuf[pl.ds(i * L, L)]
                s = srcbuf[pl.ds(i * L, L)]
                m = (d >= lo) & (d < lo + CHUNK)
                mi = m.astype(jnp.int32)
                cs = plsc.cumsum(mi)
                pos = off + cs - 1
                packed = lax.shift_left(d, 16) | s
                plsc.store_scatter(sel_pk, [pos], packed, mask=m)
                return off + cs[L - 1]

            nsel = lax.fori_loop(0, SB // L, sel_step, jnp.int32(0))

            dummy_pk = lax.shift_left(jnp.full((L,), 0, jnp.int32) + lo, 16)
            for j in range(B // L):
                ppos = nsel + j * L + iota
                plsc.store_scatter(sel_pk, [ppos], dummy_pk)
            nchunks = nsel // B + 1

            def gathers(idxsrcS, idxdstS, kbufS, qbufS, vbufS, semS):
                return (pltpu.make_async_copy(ktab.at[idxsrcS.at[0]], kbufS, semS),
                        pltpu.make_async_copy(qtab.at[idxdstS.at[0]], qbufS, semS),
                        pltpu.make_async_copy(vtab.at[idxsrcS.at[0]], vbufS, semS))

            def prefetch(c, slot):
                (idxsrcS, idxdstS, idxstageS, idxdenS,
                 kbufS, qbufS, vbufS, semS) = slot
                base = c * B
                for j in range(B // L):
                    gpos = base + j * L + iota
                    w = sel_pk[pl.ds(base + j * L, L)]
                    sv_ = w & 0xFFFF
                    dv = lax.shift_right_logical(w, 16)
                    idxsrcS[0, j * L:(j + 1) * L] = sv_
                    idxdstS[0, j * L:(j + 1) * L] = dv
                    local = jnp.where(gpos < nsel, dv - lo, DUMMY_LOCAL)
                    idxstageS[0, j * L:(j + 1) * L] = local
                    idxdenS[0, j * L:(j + 1) * L] = jnp.right_shift(local, 3)
                for cp in gathers(idxsrcS, idxdstS, kbufS, qbufS, vbufS, semS):
                    cp.start()

            def compute(slot):
                (idxsrcS, idxdstS, idxstageS, idxdenS,
                 kbufS, qbufS, vbufS, semS) = slot
                for cp in gathers(idxsrcS, idxdstS, kbufS, qbufS, vbufS, semS):
                    cp.wait()

                def edge_group(g, _):
                    locvec = idxstageS[0, pl.ds(g * L, L)]
                    for e16 in range(L):
                        e = g * L + e16
                        sv = jnp.zeros((L,), jnp.float32)
                        for h in range(H):
                            kh = kbufS[e, h * DK:(h + 1) * DK]
                            qh = qbufS[e, h * DK:(h + 1) * DK]
                            sh = plsc.cumsum(kh * qh)[L - 1]
                            sv = jnp.where(iota == h, sh, sv)
                        ex = jnp.exp(sv)
                        for blk in range(8):
                            qbufS[e, blk * L:(blk + 1) * L] = zrow
                        boff = (locvec[e16] & 7) * L
                        qbufS[e, pl.ds(boff, L)] = ex
                        for h in range(H):
                            vh = vbufS[e, h * DK:(h + 1) * DK]
                            vbufS[e, h * DK:(h + 1) * DK] = vh * ex[h]
                    return 0

                lax.fori_loop(0, B // L, edge_group, 0)

                pltpu.sync_copy(vbufS, mu_acc.at[idxstageS.at[0]], add=True)
                pltpu.sync_copy(qbufS, den_acc.at[idxdenS.at[0]], add=True)

            slot0 = (idxsrc0, idxdst0, idxstage0, idxden0,
                     kbuf0, qbuf0, vbuf0, semg0)
            slot1 = (idxsrc1, idxdst1, idxstage1, idxden1,
                     kbuf1, qbuf1, vbuf1, semg1)
            prefetch(jnp.int32(0), slot0)

            def pair_body(t, _):
                c0 = 2 * t

                @pl.when(c0 + 1 < nchunks)
                def _():
                    prefetch(c0 + 1, slot1)

                compute(slot0)

                @pl.when(c0 + 2 < nchunks)
                def _():
                    prefetch(c0 + 2, slot0)

                @pl.when(c0 + 1 < nchunks)
                def _():
                    compute(slot1)
                return 0

            lax.fori_loop(0, (nchunks + 1) // 2, pair_body, 0)
            return 0

        lax.fori_loop(0, NBLK, block_body, 0)
        plsc.subcore_barrier()

        def norm_drain(start, dstart, nrows, ndrows):
            cp = pltpu.async_copy(mu_acc.at[pl.ds(start, nrows)],
                                  kbuf0.at[pl.ds(0, nrows)], semg0)
            pltpu.sync_copy(den_acc.at[pl.ds(dstart, ndrows)],
                            qbuf0.at[pl.ds(0, ndrows)])
            cp.wait()

            def node_step(n, _):
                r = jnp.right_shift(n, 3)
                c0 = (n & 7) * L
                dvec = jnp.maximum(qbuf0[r, pl.ds(c0, L)], 1e-37)
                for h in range(H):
                    mh = kbuf0[n, h * DK:(h + 1) * DK]
                    kbuf0[n, h * DK:(h + 1) * DK] = mh / dvec[h]
                return 0

            lax.fori_loop(0, nrows, node_step, 0)
            pltpu.sync_copy(kbuf0.at[pl.ds(0, nrows)],
                            mu_out.at[pl.ds(lo + start, nrows)])

        @pl.when(sid < NS - 1)
        def _():
            for c in range(7):
                norm_drain(sid * DRAIN_Q + c * NB,
                           sid * (DRAIN_Q // 8) + c * (NB // 8), NB, NB // 8)

        @pl.when(sid == NS - 1)
        def _():
            for c in range(6):
                norm_drain((NS - 1) * DRAIN_Q + c * NB,
                           (NS - 1) * (DRAIN_Q // 8) + c * (NB // 8), NB, NB // 8)
            norm_drain((NS - 1) * DRAIN_Q + 6 * NB,
                       (NS - 1) * (DRAIN_Q // 8) + 6 * (NB // 8),
                       DRAIN_LAST_TAIL, DRAIN_LAST_TAIL // 8)
        plsc.subcore_barrier()
        return 0

    lax.fori_loop(0, NPASS, pass_body, 0)


def _run_edges(src, dst, ktab, qtab, vtab):
    mesh = plsc.VectorSubcoreMesh(core_axis_name="c", subcore_axis_name="s",
                                  num_cores=NC, num_subcores=NS)
    run = pl.kernel(
        _edge_kernel_body,
        out_type=jax.ShapeDtypeStruct((N_OUT_PAD, OUT_DIM), jnp.float32),
        mesh=mesh,
        scratch_types=[
            pltpu.VMEM_SHARED((ACC_ROWS, OUT_DIM), jnp.float32),
            pltpu.VMEM_SHARED((DEN_ROWS, OUT_DIM), jnp.float32),
            pltpu.VMEM((SB,), jnp.int32),
            pltpu.VMEM((SB,), jnp.int32),
            pltpu.VMEM((SEL_CAP,), jnp.int32),
            pltpu.VMEM((B, OUT_DIM), jnp.float32),
            pltpu.VMEM((B, OUT_DIM), jnp.float32),
            pltpu.VMEM((B, OUT_DIM), jnp.float32),
            pltpu.VMEM((B, OUT_DIM), jnp.float32),
            pltpu.VMEM((B, OUT_DIM), jnp.float32),
            pltpu.VMEM((B, OUT_DIM), jnp.float32),
            pltpu.VMEM((1, B), jnp.int32),
            pltpu.VMEM((1, B), jnp.int32),
            pltpu.VMEM((1, B), jnp.int32),
            pltpu.VMEM((1, B), jnp.int32),
            pltpu.VMEM((1, B), jnp.int32),
            pltpu.VMEM((1, B), jnp.int32),
            pltpu.VMEM((1, B), jnp.int32),
            pltpu.VMEM((1, B), jnp.int32),
            pltpu.SemaphoreType.DMA,
            pltpu.SemaphoreType.DMA,
        ],
        compiler_params=pltpu.CompilerParams(needs_layout_passes=False),
    )
    return run(src, dst, ktab, qtab, vtab)


def _final_body(muw_ref, muc_ref, tp_ref, wa_ref, ba_ref,
                g_ref, b_ref, al_ref, out_ref):
    mu = (muw_ref[...] + muc_ref[...]) * 0.5
    trans = jnp.dot(mu, wa_ref[...], preferred_element_type=jnp.float32) + ba_ref[...]
    alpha = al_ref[0, 0]
    out = trans * alpha + (1.0 - alpha) * tp_ref[...]
    mean = jnp.mean(out, axis=1, keepdims=True)
    d = out - mean
    var = jnp.mean(d * d, axis=1, keepdims=True)
    out_ref[...] = d * lax.rsqrt(var + 1e-5) * g_ref[...] + b_ref[...]


def _run_final(mu_w, mu_c, t_paper, waT, ba1, g1, b1, alpha):
    row_spec = pl.BlockSpec((BR, OUT_DIM), lambda i: (i, 0))
    w_spec = pl.BlockSpec((OUT_DIM, OUT_DIM), lambda i: (0, 0))
    b_spec = pl.BlockSpec((1, OUT_DIM), lambda i: (0, 0))
    s_spec = pl.BlockSpec((1, 1), lambda i: (0, 0))
    return pl.pallas_call(
        _final_body,
        grid=(GRID,),
        in_specs=[row_spec, row_spec, row_spec,
                  w_spec, b_spec, b_spec, b_spec, s_spec],
        out_specs=row_spec,
        out_shape=jax.ShapeDtypeStruct((N_NODE, OUT_DIM), jnp.float32),
    )(mu_w, mu_c, t_paper, waT, ba1, g1, b1, alpha)


def kernel(h_author, h_paper, t_author, t_paper, edge_index_writes,
           edge_index_cites, Wk, bk, Wq, bq, Wv, bv, Wa, ba, ln_scale,
           ln_bias, relation_pri, relation_att, relation_msg, skip):
    f32 = jnp.float32

    def fold(WT, bvec, rel, scale):
        wt = jnp.einsum('dhi,hij->dhj', WT.reshape(IN_DIM, H, DK), rel)
        bf = jnp.einsum('hi,hij->hj', bvec.reshape(H, DK), rel)
        wt = wt * scale[None, :, None]
        bf = bf * scale[:, None]
        return wt.reshape(IN_DIM, OUT_DIM), bf.reshape(1, OUT_DIM)

    pri_w = relation_pri[0] / SQRT_DK
    pri_c = relation_pri[1] / SQRT_DK
    ones_h = jnp.ones((H,), f32)
    wkw, bkw = fold(Wk[0].T, bk[0], relation_att[0], pri_w)
    wkc, bkc = fold(Wk[1].T, bk[1], relation_att[1], pri_c)
    wvw, bvw = fold(Wv[0].T, bv[0], relation_msg[0], ones_h)
    wvc, bvc = fold(Wv[1].T, bv[1], relation_msg[1], ones_h)
    wq = Wq[1].T
    bq1 = bq[1].reshape(1, OUT_DIM)

    k_w, v_w, k_c, v_c, q = _run_projections(
        h_author, h_paper, t_paper, wkw, wvw, wkc, wvc, wq,
        bkw, bvw, bkc, bvc, bq1)

    def pad_edges(ei):
        src = jnp.concatenate([ei[0], jnp.zeros((E_PAD - E,), jnp.int32)])
        dst = jnp.concatenate([ei[1], jnp.full((E_PAD - E,), -1, jnp.int32)])
        return src, dst

    src_w, dst_w = pad_edges(edge_index_writes)
    src_c, dst_c = pad_edges(edge_index_cites)

    mu_w = _run_edges(src_w, dst_w, k_w, q, v_w)
    mu_c = _run_edges(src_c, dst_c, k_c, q, v_c)

    alpha = jax.nn.sigmoid(skip[1]).reshape(1, 1).astype(f32)
    return _run_final(mu_w, mu_c, t_paper, Wa[1].T,
                      ba[1].reshape(1, OUT_DIM), ln_scale[1].reshape(1, OUT_DIM),
                      ln_bias[1].reshape(1, OUT_DIM), alpha)
